# baseline (speedup 1.0000x reference)
import functools

import jax
import jax.numpy as jnp
from jax import lax
from jax.experimental import pallas as pl
from jax.experimental.pallas import tpu as pltpu
from jax.experimental.pallas import tpu_sc as plsc

N = 20000
NPAD = 20480
PADVAL = 1.0e6
BI = 256
BJ = 2048
NI = NPAD // BI
NJ = NPAD // BJ

NWORKERS = 32
BPW = NPAD // NWORKERS


def _knn_body(pred_ref, ttar_ref, rowidx_ref, colidx_ref,
              rmin_scr, cmin_scr, cidx_scr):
    i = pl.program_id(0)
    j = pl.program_id(1)
    a = pred_ref[...]
    t = ttar_ref[...]
    a0 = a[:, 0:1]
    a1 = a[:, 1:2]
    a2 = a[:, 2:3]
    b0 = t[0:1, :]
    b1 = t[1:2, :]
    b2 = t[2:3, :]
    bf = jnp.bfloat16
    f32 = jnp.float32
    a0b = a0.astype(bf).astype(f32)
    a1b = a1.astype(bf).astype(f32)
    a2b = a2.astype(bf).astype(f32)
    b0b = b0.astype(bf).astype(f32)
    b1b = b1.astype(bf).astype(f32)
    b2b = b2.astype(bf).astype(f32)
    a_sq = (a0 * a0 + a1 * a1) + a2 * a2
    b_sq = (b0 * b0 + b1 * b1) + b2 * b2
    dot = (a0b * b0b + a1b * b1b) + a2b * b2b
    s = (a_sq - 2.0 * dot) + b_sq

    tmin = jnp.min(s, axis=1, keepdims=True)
    lane = lax.broadcasted_iota(jnp.int32, (BI, BJ), 1)
    jsel = jnp.where(s == tmin, lane, NPAD)
    targ = jnp.min(jsel, axis=1, keepdims=True) + j * BJ

    cmin = jnp.min(s, axis=0, keepdims=True)
    row = lax.broadcasted_iota(jnp.int32, (BI, BJ), 0)
    isel = jnp.where(s == cmin, row, NPAD)
    carg = jnp.min(isel, axis=0, keepdims=True) + i * BI

    @pl.when(j == 0)
    def _():
        rmin_scr[...] = tmin
        rowidx_ref[...] = targ

    @pl.when(j > 0)
    def _():
        prev = rmin_scr[...]
        imp = tmin < prev
        rmin_scr[...] = jnp.where(imp, tmin, prev)
        rowidx_ref[...] = jnp.where(imp, targ, rowidx_ref[...])

    @pl.when(i == 0)
    def _():
        cmin_scr[0:1, pl.ds(j * BJ, BJ)] = cmin
        cidx_scr[0:1, pl.ds(j * BJ, BJ)] = carg

    @pl.when(i > 0)
    def _():
        pc = cmin_scr[0:1, pl.ds(j * BJ, BJ)]
        imp = cmin < pc
        cmin_scr[0:1, pl.ds(j * BJ, BJ)] = jnp.where(imp, cmin, pc)
        cidx_scr[0:1, pl.ds(j * BJ, BJ)] = jnp.where(
            imp, carg, cidx_scr[0:1, pl.ds(j * BJ, BJ)])

    @pl.when(i == NI - 1)
    def _():
        colidx_ref[...] = cidx_scr[0:1, pl.ds(j * BJ, BJ)]


_knn_call = pl.pallas_call(
    _knn_body,
    grid=(NI, NJ),
    in_specs=[
        pl.BlockSpec((BI, 4), lambda i, j: (i, 0)),
        pl.BlockSpec((4, BJ), lambda i, j: (0, j)),
    ],
    out_specs=[
        pl.BlockSpec((BI, 1), lambda i, j: (i, 0)),
        pl.BlockSpec((1, BJ), lambda i, j: (0, j)),
    ],
    out_shape=[
        jax.ShapeDtypeStruct((NPAD, 1), jnp.int32),
        jax.ShapeDtypeStruct((1, NPAD), jnp.int32),
    ],
    scratch_shapes=[
        pltpu.VMEM((BI, 1), jnp.float32),
        pltpu.VMEM((1, NPAD), jnp.float32),
        pltpu.VMEM((1, NPAD), jnp.int32),
    ],
    compiler_params=pltpu.CompilerParams(
        dimension_semantics=("arbitrary", "arbitrary")),
)


CHUNK_SC = 128
DROW = 16


def _sc_gather_body(tab_t, idx_t, tab_p, idx_p, out_t, out_p,
                    idx_v, rows_v, sem):
    wid = lax.axis_index("s") * 2 + lax.axis_index("c")
    base = wid * BPW
    for c in range(BPW // CHUNK_SC):
        off = base + c * CHUNK_SC
        pltpu.sync_copy(idx_t.at[pl.ds(off, CHUNK_SC)], idx_v)
        pltpu.async_copy(tab_t.at[idx_v], rows_v, sem).wait()
        pltpu.sync_copy(rows_v, out_t.at[pl.ds(off, CHUNK_SC)])
    for c in range(BPW // CHUNK_SC):
        off = base + c * CHUNK_SC
        pltpu.sync_copy(idx_p.at[pl.ds(off, CHUNK_SC)], idx_v)
        pltpu.async_copy(tab_p.at[idx_v], rows_v, sem).wait()
        pltpu.sync_copy(rows_v, out_p.at[pl.ds(off, CHUNK_SC)])


def _sc_gather_call(tab_t, idx_t, tab_p, idx_p):
    run = functools.partial(
        pl.kernel,
        mesh=plsc.VectorSubcoreMesh(core_axis_name="c", subcore_axis_name="s"),
        out_type=[
            jax.ShapeDtypeStruct((NPAD, DROW), jnp.float32),
            jax.ShapeDtypeStruct((NPAD, DROW), jnp.float32),
        ],
        scratch_types=[
            pltpu.VMEM((CHUNK_SC,), jnp.int32),
            pltpu.VMEM((CHUNK_SC, DROW), jnp.float32),
            pltpu.SemaphoreType.DMA,
        ],
        compiler_params=pltpu.CompilerParams(use_tc_tiling_on_sc=False),
    )(_sc_gather_body)
    return run(tab_t, idx_t, tab_p, idx_p)


_R, _C = NPAD // 128, 128
_NPLANES = 14


def _combine_body(stk_ref, out_ref):
    def plane(k):
        return stk_ref[k]

    p0, p1, p2, pint = plane(0), plane(1), plane(2), plane(3)
    g0, g1, g2, gint = plane(4), plane(5), plane(6), plane(7)
    t0, t1, t2 = plane(8), plane(9), plane(10)
    h0, h1, h2 = plane(11), plane(12), plane(13)
    r = lax.broadcasted_iota(jnp.int32, (_R, _C), 0)
    c = lax.broadcasted_iota(jnp.int32, (_R, _C), 1)
    valid = (r * _C + c) < N
    d1 = jnp.sqrt(((p0 - g0) * (p0 - g0) + (p1 - g1) * (p1 - g1))
                  + (p2 - g2) * (p2 - g2))
    d2 = jnp.sqrt(((t0 - h0) * (t0 - h0) + (t1 - h1) * (t1 - h1))
                  + (t2 - h2) * (t2 - h2))
    s1 = jnp.sum(jnp.where(valid, d1, 0.0))
    s2 = jnp.sum(jnp.where(valid, d2, 0.0))
    si = jnp.sum(jnp.where(valid, (pint - gint) * (pint - gint), 0.0))
    total = (s1 + s2) / N + 0.5 * (si / N)
    out_ref[...] = jnp.reshape(total, (1, 1))


_combine_call = pl.pallas_call(
    _combine_body,
    out_shape=jax.ShapeDtypeStruct((1, 1), jnp.float32),
)


def kernel(pred, target):
    pred_p = jnp.pad(pred, ((0, NPAD - N), (0, 0)), constant_values=PADVAL)
    targ_p = jnp.pad(target, ((0, NPAD - N), (0, 0)), constant_values=PADVAL)
    ttar = targ_p.T

    rowidx, colidx = _knn_call(pred_p, ttar)

    tab_t = jnp.pad(targ_p, ((0, 0), (0, DROW - 4)))
    tab_p = jnp.pad(pred_p, ((0, 0), (0, DROW - 4)))
    g_t, g_p = _sc_gather_call(tab_t, rowidx.reshape(NPAD),
                               tab_p, colidx.reshape(NPAD))

    def pl2(x):
        return x.reshape(_R, _C)

    stk = jnp.stack([
        pl2(pred_p[:, 0]), pl2(pred_p[:, 1]), pl2(pred_p[:, 2]),
        pl2(pred_p[:, 3]),
        pl2(g_t[:, 0]), pl2(g_t[:, 1]), pl2(g_t[:, 2]), pl2(g_t[:, 3]),
        pl2(targ_p[:, 0]), pl2(targ_p[:, 1]), pl2(targ_p[:, 2]),
        pl2(g_p[:, 0]), pl2(g_p[:, 1]), pl2(g_p[:, 2]),
    ])
    total = _combine_call(stk)
    return total[0, 0]


# speedup vs baseline: 1.1876x; 1.1876x over previous
import functools

import jax
import jax.numpy as jnp
from jax import lax
from jax.experimental import pallas as pl
from jax.experimental.pallas import tpu as pltpu
from jax.experimental.pallas import tpu_sc as plsc

N = 20000
NPAD = 20480
PADVAL = 1.0e6
BI = 256
BJ = 2048
NI = NPAD // BI
NJ = NPAD // BJ

NWORKERS = 32
BPW = NPAD // NWORKERS


def _knn_body(pred_ref, ttar_ref, rowidx_ref, colidx_ref,
              rmin_scr, cmin_scr, cidx_scr):
    i = pl.program_id(0)
    j = pl.program_id(1)
    a = pred_ref[...]
    t = ttar_ref[...]
    a0 = a[:, 0:1]
    a1 = a[:, 1:2]
    a2 = a[:, 2:3]
    b0 = t[0:1, :]
    b1 = t[1:2, :]
    b2 = t[2:3, :]
    bf = jnp.bfloat16
    f32 = jnp.float32
    a_sq = (a0 * a0 + a1 * a1) + a2 * a2
    b_sq = (b0 * b0 + b1 * b1) + b2 * b2
    dot = lax.dot_general(a[:, 0:3].astype(bf), t[0:3, :].astype(bf),
                          (((1,), (0,)), ((), ())),
                          preferred_element_type=f32)
    s = (a_sq - 2.0 * dot) + b_sq

    tmin = jnp.min(s, axis=1, keepdims=True)
    lane = lax.broadcasted_iota(jnp.int32, (BI, BJ), 1)
    jsel = jnp.where(s == tmin, lane, NPAD)
    targ = jnp.min(jsel, axis=1, keepdims=True) + j * BJ

    cmin = jnp.min(s, axis=0, keepdims=True)
    row = lax.broadcasted_iota(jnp.int32, (BI, BJ), 0)
    isel = jnp.where(s == cmin, row, NPAD)
    carg = jnp.min(isel, axis=0, keepdims=True) + i * BI

    @pl.when(j == 0)
    def _():
        rmin_scr[...] = tmin
        rowidx_ref[...] = targ

    @pl.when(j > 0)
    def _():
        prev = rmin_scr[...]
        imp = tmin < prev
        rmin_scr[...] = jnp.where(imp, tmin, prev)
        rowidx_ref[...] = jnp.where(imp, targ, rowidx_ref[...])

    @pl.when(i == 0)
    def _():
        cmin_scr[0:1, pl.ds(j * BJ, BJ)] = cmin
        cidx_scr[0:1, pl.ds(j * BJ, BJ)] = carg

    @pl.when(i > 0)
    def _():
        pc = cmin_scr[0:1, pl.ds(j * BJ, BJ)]
        imp = cmin < pc
        cmin_scr[0:1, pl.ds(j * BJ, BJ)] = jnp.where(imp, cmin, pc)
        cidx_scr[0:1, pl.ds(j * BJ, BJ)] = jnp.where(
            imp, carg, cidx_scr[0:1, pl.ds(j * BJ, BJ)])

    @pl.when(i == NI - 1)
    def _():
        colidx_ref[...] = cidx_scr[0:1, pl.ds(j * BJ, BJ)]


_knn_call = pl.pallas_call(
    _knn_body,
    grid=(NI, NJ),
    in_specs=[
        pl.BlockSpec((BI, 4), lambda i, j: (i, 0)),
        pl.BlockSpec((4, BJ), lambda i, j: (0, j)),
    ],
    out_specs=[
        pl.BlockSpec((BI, 1), lambda i, j: (i, 0)),
        pl.BlockSpec((1, BJ), lambda i, j: (0, j)),
    ],
    out_shape=[
        jax.ShapeDtypeStruct((NPAD, 1), jnp.int32),
        jax.ShapeDtypeStruct((1, NPAD), jnp.int32),
    ],
    scratch_shapes=[
        pltpu.VMEM((BI, 1), jnp.float32),
        pltpu.VMEM((1, NPAD), jnp.float32),
        pltpu.VMEM((1, NPAD), jnp.int32),
    ],
    compiler_params=pltpu.CompilerParams(
        dimension_semantics=("arbitrary", "arbitrary")),
)


CHUNK_SC = 128
DROW = 16


def _sc_gather_body(tab_t, idx_t, tab_p, idx_p, out_t, out_p,
                    idx_v, rows_v, sem):
    wid = lax.axis_index("s") * 2 + lax.axis_index("c")
    base = wid * BPW
    for c in range(BPW // CHUNK_SC):
        off = base + c * CHUNK_SC
        pltpu.sync_copy(idx_t.at[pl.ds(off, CHUNK_SC)], idx_v)
        pltpu.async_copy(tab_t.at[idx_v], rows_v, sem).wait()
        pltpu.sync_copy(rows_v, out_t.at[pl.ds(off, CHUNK_SC)])
    for c in range(BPW // CHUNK_SC):
        off = base + c * CHUNK_SC
        pltpu.sync_copy(idx_p.at[pl.ds(off, CHUNK_SC)], idx_v)
        pltpu.async_copy(tab_p.at[idx_v], rows_v, sem).wait()
        pltpu.sync_copy(rows_v, out_p.at[pl.ds(off, CHUNK_SC)])


def _sc_gather_call(tab_t, idx_t, tab_p, idx_p):
    run = functools.partial(
        pl.kernel,
        mesh=plsc.VectorSubcoreMesh(core_axis_name="c", subcore_axis_name="s"),
        out_type=[
            jax.ShapeDtypeStruct((NPAD, DROW), jnp.float32),
            jax.ShapeDtypeStruct((NPAD, DROW), jnp.float32),
        ],
        scratch_types=[
            pltpu.VMEM((CHUNK_SC,), jnp.int32),
            pltpu.VMEM((CHUNK_SC, DROW), jnp.float32),
            pltpu.SemaphoreType.DMA,
        ],
        compiler_params=pltpu.CompilerParams(use_tc_tiling_on_sc=False),
    )(_sc_gather_body)
    return run(tab_t, idx_t, tab_p, idx_p)


_R, _C = NPAD // 128, 128
_NPLANES = 14


def _combine_body(stk_ref, out_ref):
    def plane(k):
        return stk_ref[k]

    p0, p1, p2, pint = plane(0), plane(1), plane(2), plane(3)
    g0, g1, g2, gint = plane(4), plane(5), plane(6), plane(7)
    t0, t1, t2 = plane(8), plane(9), plane(10)
    h0, h1, h2 = plane(11), plane(12), plane(13)
    r = lax.broadcasted_iota(jnp.int32, (_R, _C), 0)
    c = lax.broadcasted_iota(jnp.int32, (_R, _C), 1)
    valid = (r * _C + c) < N
    d1 = jnp.sqrt(((p0 - g0) * (p0 - g0) + (p1 - g1) * (p1 - g1))
                  + (p2 - g2) * (p2 - g2))
    d2 = jnp.sqrt(((t0 - h0) * (t0 - h0) + (t1 - h1) * (t1 - h1))
                  + (t2 - h2) * (t2 - h2))
    s1 = jnp.sum(jnp.where(valid, d1, 0.0))
    s2 = jnp.sum(jnp.where(valid, d2, 0.0))
    si = jnp.sum(jnp.where(valid, (pint - gint) * (pint - gint), 0.0))
    total = (s1 + s2) / N + 0.5 * (si / N)
    out_ref[...] = jnp.reshape(total, (1, 1))


_combine_call = pl.pallas_call(
    _combine_body,
    out_shape=jax.ShapeDtypeStruct((1, 1), jnp.float32),
)


def kernel(pred, target):
    pred_p = jnp.pad(pred, ((0, NPAD - N), (0, 0)), constant_values=PADVAL)
    targ_p = jnp.pad(target, ((0, NPAD - N), (0, 0)), constant_values=PADVAL)
    ttar = targ_p.T

    rowidx, colidx = _knn_call(pred_p, ttar)

    tab_t = jnp.pad(targ_p, ((0, 0), (0, DROW - 4)))
    tab_p = jnp.pad(pred_p, ((0, 0), (0, DROW - 4)))
    g_t, g_p = _sc_gather_call(tab_t, rowidx.reshape(NPAD),
                               tab_p, colidx.reshape(NPAD))

    def pl2(x):
        return x.reshape(_R, _C)

    stk = jnp.stack([
        pl2(pred_p[:, 0]), pl2(pred_p[:, 1]), pl2(pred_p[:, 2]),
        pl2(pred_p[:, 3]),
        pl2(g_t[:, 0]), pl2(g_t[:, 1]), pl2(g_t[:, 2]), pl2(g_t[:, 3]),
        pl2(targ_p[:, 0]), pl2(targ_p[:, 1]), pl2(targ_p[:, 2]),
        pl2(g_p[:, 0]), pl2(g_p[:, 1]), pl2(g_p[:, 2]),
    ])
    total = _combine_call(stk)
    return total[0, 0]


# speedup vs baseline: 1.4595x; 1.2289x over previous
import functools

import jax
import jax.numpy as jnp
from jax import lax
from jax.experimental import pallas as pl
from jax.experimental.pallas import tpu as pltpu
from jax.experimental.pallas import tpu_sc as plsc

N = 20000
NPAD = 20480
PADVAL = 1.0e6
BI = 256
BJ = 2048
NI = NPAD // BI
NJ = NPAD // BJ

NWORKERS = 32
BPW = NPAD // NWORKERS


def _knn_body(pred_ref, ttar_ref, rowidx_ref, colidx_ref,
              rmin_scr, cmin_scr, cidx_scr):
    i = pl.program_id(0)
    j = pl.program_id(1)
    a = pred_ref[...]
    t = ttar_ref[...]
    a0 = a[:, 0:1]
    a1 = a[:, 1:2]
    a2 = a[:, 2:3]
    b0 = t[0:1, :]
    b1 = t[1:2, :]
    b2 = t[2:3, :]
    bf = jnp.bfloat16
    f32 = jnp.float32
    i32 = jnp.int32
    a_sq = (a0 * a0 + a1 * a1) + a2 * a2
    b_sq = (b0 * b0 + b1 * b1) + b2 * b2
    dotm2 = lax.dot_general((a[:, 0:3] * -2.0).astype(bf), t[0:3, :].astype(bf),
                            (((1,), (0,)), ((), ())),
                            preferred_element_type=f32)
    s = (a_sq + dotm2) + b_sq

    v = s[:, 0:128]
    kk = jnp.zeros((BI, 128), i32)
    for k in range(1, BJ // 128):
        sk = s[:, k * 128:(k + 1) * 128]
        m = sk < v
        v = jnp.where(m, sk, v)
        kk = jnp.where(m, k, kk)
    tmin = jnp.min(v, axis=1, keepdims=True)
    lane = lax.broadcasted_iota(i32, (BI, 128), 1)
    cand = kk * 128 + lane
    jfull = jnp.where(v == tmin, cand, NPAD)
    targ = jnp.min(jfull, axis=1, keepdims=True) + j * BJ

    w = s[0:8, :]
    rr = jnp.zeros((8, BJ), i32)
    for r in range(1, BI // 8):
        sr_ = s[r * 8:(r + 1) * 8, :]
        m = sr_ < w
        w = jnp.where(m, sr_, w)
        rr = jnp.where(m, r, rr)
    cmin = jnp.min(w, axis=0, keepdims=True)
    sub = lax.broadcasted_iota(i32, (8, BJ), 0)
    cand2 = rr * 8 + sub
    ifull = jnp.where(w == cmin, cand2, NPAD)
    carg = jnp.min(ifull, axis=0, keepdims=True) + i * BI

    @pl.when(j == 0)
    def _():
        rmin_scr[...] = tmin
        rowidx_ref[...] = targ

    @pl.when(j > 0)
    def _():
        prev = rmin_scr[...]
        imp = tmin < prev
        rmin_scr[...] = jnp.where(imp, tmin, prev)
        rowidx_ref[...] = jnp.where(imp, targ, rowidx_ref[...])

    @pl.when(i == 0)
    def _():
        cmin_scr[0:1, pl.ds(j * BJ, BJ)] = cmin
        cidx_scr[0:1, pl.ds(j * BJ, BJ)] = carg

    @pl.when(i > 0)
    def _():
        pc = cmin_scr[0:1, pl.ds(j * BJ, BJ)]
        imp = cmin < pc
        cmin_scr[0:1, pl.ds(j * BJ, BJ)] = jnp.where(imp, cmin, pc)
        cidx_scr[0:1, pl.ds(j * BJ, BJ)] = jnp.where(
            imp, carg, cidx_scr[0:1, pl.ds(j * BJ, BJ)])

    @pl.when(i == NI - 1)
    def _():
        colidx_ref[...] = cidx_scr[0:1, pl.ds(j * BJ, BJ)]


_knn_call = pl.pallas_call(
    _knn_body,
    grid=(NI, NJ),
    in_specs=[
        pl.BlockSpec((BI, 4), lambda i, j: (i, 0)),
        pl.BlockSpec((4, BJ), lambda i, j: (0, j)),
    ],
    out_specs=[
        pl.BlockSpec((BI, 1), lambda i, j: (i, 0)),
        pl.BlockSpec((1, BJ), lambda i, j: (0, j)),
    ],
    out_shape=[
        jax.ShapeDtypeStruct((NPAD, 1), jnp.int32),
        jax.ShapeDtypeStruct((1, NPAD), jnp.int32),
    ],
    scratch_shapes=[
        pltpu.VMEM((BI, 1), jnp.float32),
        pltpu.VMEM((1, NPAD), jnp.float32),
        pltpu.VMEM((1, NPAD), jnp.int32),
    ],
    compiler_params=pltpu.CompilerParams(
        dimension_semantics=("arbitrary", "arbitrary")),
)


CHUNK_SC = 128
DROW = 16


def _sc_gather_body(tab_t, idx_t, tab_p, idx_p, out_t, out_p,
                    idx_v, rows_v, sem):
    wid = lax.axis_index("s") * 2 + lax.axis_index("c")
    base = wid * BPW
    for c in range(BPW // CHUNK_SC):
        off = base + c * CHUNK_SC
        pltpu.sync_copy(idx_t.at[pl.ds(off, CHUNK_SC)], idx_v)
        pltpu.async_copy(tab_t.at[idx_v], rows_v, sem).wait()
        pltpu.sync_copy(rows_v, out_t.at[pl.ds(off, CHUNK_SC)])
    for c in range(BPW // CHUNK_SC):
        off = base + c * CHUNK_SC
        pltpu.sync_copy(idx_p.at[pl.ds(off, CHUNK_SC)], idx_v)
        pltpu.async_copy(tab_p.at[idx_v], rows_v, sem).wait()
        pltpu.sync_copy(rows_v, out_p.at[pl.ds(off, CHUNK_SC)])


def _sc_gather_call(tab_t, idx_t, tab_p, idx_p):
    run = functools.partial(
        pl.kernel,
        mesh=plsc.VectorSubcoreMesh(core_axis_name="c", subcore_axis_name="s"),
        out_type=[
            jax.ShapeDtypeStruct((NPAD, DROW), jnp.float32),
            jax.ShapeDtypeStruct((NPAD, DROW), jnp.float32),
        ],
        scratch_types=[
            pltpu.VMEM((CHUNK_SC,), jnp.int32),
            pltpu.VMEM((CHUNK_SC, DROW), jnp.float32),
            pltpu.SemaphoreType.DMA,
        ],
        compiler_params=pltpu.CompilerParams(use_tc_tiling_on_sc=False),
    )(_sc_gather_body)
    return run(tab_t, idx_t, tab_p, idx_p)


_R, _C = NPAD // 128, 128
_NPLANES = 14


def _combine_body(stk_ref, out_ref):
    def plane(k):
        return stk_ref[k]

    p0, p1, p2, pint = plane(0), plane(1), plane(2), plane(3)
    g0, g1, g2, gint = plane(4), plane(5), plane(6), plane(7)
    t0, t1, t2 = plane(8), plane(9), plane(10)
    h0, h1, h2 = plane(11), plane(12), plane(13)
    r = lax.broadcasted_iota(jnp.int32, (_R, _C), 0)
    c = lax.broadcasted_iota(jnp.int32, (_R, _C), 1)
    valid = (r * _C + c) < N
    d1 = jnp.sqrt(((p0 - g0) * (p0 - g0) + (p1 - g1) * (p1 - g1))
                  + (p2 - g2) * (p2 - g2))
    d2 = jnp.sqrt(((t0 - h0) * (t0 - h0) + (t1 - h1) * (t1 - h1))
                  + (t2 - h2) * (t2 - h2))
    s1 = jnp.sum(jnp.where(valid, d1, 0.0))
    s2 = jnp.sum(jnp.where(valid, d2, 0.0))
    si = jnp.sum(jnp.where(valid, (pint - gint) * (pint - gint), 0.0))
    total = (s1 + s2) / N + 0.5 * (si / N)
    out_ref[...] = jnp.reshape(total, (1, 1))


_combine_call = pl.pallas_call(
    _combine_body,
    out_shape=jax.ShapeDtypeStruct((1, 1), jnp.float32),
)


def kernel(pred, target):
    pred_p = jnp.pad(pred, ((0, NPAD - N), (0, 0)), constant_values=PADVAL)
    targ_p = jnp.pad(target, ((0, NPAD - N), (0, 0)), constant_values=PADVAL)
    ttar = targ_p.T

    rowidx, colidx = _knn_call(pred_p, ttar)

    tab_t = jnp.pad(targ_p, ((0, 0), (0, DROW - 4)))
    tab_p = jnp.pad(pred_p, ((0, 0), (0, DROW - 4)))
    g_t, g_p = _sc_gather_call(tab_t, rowidx.reshape(NPAD),
                               tab_p, colidx.reshape(NPAD))

    def pl2(x):
        return x.reshape(_R, _C)

    stk = jnp.stack([
        pl2(pred_p[:, 0]), pl2(pred_p[:, 1]), pl2(pred_p[:, 2]),
        pl2(pred_p[:, 3]),
        pl2(g_t[:, 0]), pl2(g_t[:, 1]), pl2(g_t[:, 2]), pl2(g_t[:, 3]),
        pl2(targ_p[:, 0]), pl2(targ_p[:, 1]), pl2(targ_p[:, 2]),
        pl2(g_p[:, 0]), pl2(g_p[:, 1]), pl2(g_p[:, 2]),
    ])
    total = _combine_call(stk)
    return total[0, 0]


# speedup vs baseline: 1.5036x; 1.0302x over previous
import functools

import jax
import jax.numpy as jnp
from jax import lax
from jax.experimental import pallas as pl
from jax.experimental.pallas import tpu as pltpu
from jax.experimental.pallas import tpu_sc as plsc

N = 20000
NPAD = 20480
PADVAL = 1.0e6
BI = 256
BJ = 2048
NI = NPAD // BI
NJ = NPAD // BJ

NWORKERS = 32
BPW = NPAD // NWORKERS


def _knn_body(pred_ref, ttar_ref, rowidx_ref, colidx_ref,
              rmin_scr, cmin_scr, cidx_scr):
    i = pl.program_id(0)
    j = pl.program_id(1)
    a = pred_ref[...]
    t = ttar_ref[...]
    f32 = jnp.float32
    i32 = jnp.int32
    s = lax.dot_general(a, t, (((1,), (0,)), ((), ())),
                        preferred_element_type=f32)

    v = s[:, 0:128]
    kk = jnp.zeros((BI, 128), i32)
    for k in range(1, BJ // 128):
        sk = s[:, k * 128:(k + 1) * 128]
        m = sk < v
        v = jnp.where(m, sk, v)
        kk = jnp.where(m, k, kk)
    tmin = jnp.min(v, axis=1, keepdims=True)
    lane = lax.broadcasted_iota(i32, (BI, 128), 1)
    cand = kk * 128 + lane
    jfull = jnp.where(v == tmin, cand, NPAD)
    targ = jnp.min(jfull, axis=1, keepdims=True) + j * BJ

    w = s[0:8, :]
    rr = jnp.zeros((8, BJ), i32)
    for r in range(1, BI // 8):
        sr_ = s[r * 8:(r + 1) * 8, :]
        m = sr_ < w
        w = jnp.where(m, sr_, w)
        rr = jnp.where(m, r, rr)
    cmin = jnp.min(w, axis=0, keepdims=True)
    sub = lax.broadcasted_iota(i32, (8, BJ), 0)
    cand2 = rr * 8 + sub
    ifull = jnp.where(w == cmin, cand2, NPAD)
    carg = jnp.min(ifull, axis=0, keepdims=True) + i * BI

    @pl.when(j == 0)
    def _():
        rmin_scr[...] = tmin
        rowidx_ref[...] = targ

    @pl.when(j > 0)
    def _():
        prev = rmin_scr[...]
        imp = tmin < prev
        rmin_scr[...] = jnp.where(imp, tmin, prev)
        rowidx_ref[...] = jnp.where(imp, targ, rowidx_ref[...])

    @pl.when(i == 0)
    def _():
        cmin_scr[0:1, pl.ds(j * BJ, BJ)] = cmin
        cidx_scr[0:1, pl.ds(j * BJ, BJ)] = carg

    @pl.when(i > 0)
    def _():
        pc = cmin_scr[0:1, pl.ds(j * BJ, BJ)]
        imp = cmin < pc
        cmin_scr[0:1, pl.ds(j * BJ, BJ)] = jnp.where(imp, cmin, pc)
        cidx_scr[0:1, pl.ds(j * BJ, BJ)] = jnp.where(
            imp, carg, cidx_scr[0:1, pl.ds(j * BJ, BJ)])

    @pl.when(i == NI - 1)
    def _():
        colidx_ref[...] = cidx_scr[0:1, pl.ds(j * BJ, BJ)]


_knn_call = pl.pallas_call(
    _knn_body,
    grid=(NI, NJ),
    in_specs=[
        pl.BlockSpec((BI, 16), lambda i, j: (i, 0)),
        pl.BlockSpec((16, BJ), lambda i, j: (0, j)),
    ],
    out_specs=[
        pl.BlockSpec((BI, 1), lambda i, j: (i, 0)),
        pl.BlockSpec((1, BJ), lambda i, j: (0, j)),
    ],
    out_shape=[
        jax.ShapeDtypeStruct((NPAD, 1), jnp.int32),
        jax.ShapeDtypeStruct((1, NPAD), jnp.int32),
    ],
    scratch_shapes=[
        pltpu.VMEM((BI, 1), jnp.float32),
        pltpu.VMEM((1, NPAD), jnp.float32),
        pltpu.VMEM((1, NPAD), jnp.int32),
    ],
    compiler_params=pltpu.CompilerParams(
        dimension_semantics=("arbitrary", "arbitrary")),
)


CHUNK_SC = 128
DROW = 16


def _sc_gather_body(tab_t, idx_t, tab_p, idx_p, out_t, out_p,
                    idx_v, rows_v, sem):
    wid = lax.axis_index("s") * 2 + lax.axis_index("c")
    base = wid * BPW
    for c in range(BPW // CHUNK_SC):
        off = base + c * CHUNK_SC
        pltpu.sync_copy(idx_t.at[pl.ds(off, CHUNK_SC)], idx_v)
        pltpu.async_copy(tab_t.at[idx_v], rows_v, sem).wait()
        pltpu.sync_copy(rows_v, out_t.at[pl.ds(off, CHUNK_SC)])
    for c in range(BPW // CHUNK_SC):
        off = base + c * CHUNK_SC
        pltpu.sync_copy(idx_p.at[pl.ds(off, CHUNK_SC)], idx_v)
        pltpu.async_copy(tab_p.at[idx_v], rows_v, sem).wait()
        pltpu.sync_copy(rows_v, out_p.at[pl.ds(off, CHUNK_SC)])


def _sc_gather_call(tab_t, idx_t, tab_p, idx_p):
    run = functools.partial(
        pl.kernel,
        mesh=plsc.VectorSubcoreMesh(core_axis_name="c", subcore_axis_name="s"),
        out_type=[
            jax.ShapeDtypeStruct((NPAD, DROW), jnp.float32),
            jax.ShapeDtypeStruct((NPAD, DROW), jnp.float32),
        ],
        scratch_types=[
            pltpu.VMEM((CHUNK_SC,), jnp.int32),
            pltpu.VMEM((CHUNK_SC, DROW), jnp.float32),
            pltpu.SemaphoreType.DMA,
        ],
        compiler_params=pltpu.CompilerParams(use_tc_tiling_on_sc=False),
    )(_sc_gather_body)
    return run(tab_t, idx_t, tab_p, idx_p)


_R, _C = NPAD // 128, 128
_NPLANES = 14


def _combine_body(stk_ref, out_ref):
    def plane(k):
        return stk_ref[k]

    p0, p1, p2, pint = plane(0), plane(1), plane(2), plane(3)
    g0, g1, g2, gint = plane(4), plane(5), plane(6), plane(7)
    t0, t1, t2 = plane(8), plane(9), plane(10)
    h0, h1, h2 = plane(11), plane(12), plane(13)
    r = lax.broadcasted_iota(jnp.int32, (_R, _C), 0)
    c = lax.broadcasted_iota(jnp.int32, (_R, _C), 1)
    valid = (r * _C + c) < N
    d1 = jnp.sqrt(((p0 - g0) * (p0 - g0) + (p1 - g1) * (p1 - g1))
                  + (p2 - g2) * (p2 - g2))
    d2 = jnp.sqrt(((t0 - h0) * (t0 - h0) + (t1 - h1) * (t1 - h1))
                  + (t2 - h2) * (t2 - h2))
    s1 = jnp.sum(jnp.where(valid, d1, 0.0))
    s2 = jnp.sum(jnp.where(valid, d2, 0.0))
    si = jnp.sum(jnp.where(valid, (pint - gint) * (pint - gint), 0.0))
    total = (s1 + s2) / N + 0.5 * (si / N)
    out_ref[...] = jnp.reshape(total, (1, 1))


_combine_call = pl.pallas_call(
    _combine_body,
    out_shape=jax.ShapeDtypeStruct((1, 1), jnp.float32),
)


def _split3(x):
    bf = jnp.bfloat16
    f32 = jnp.float32
    hi = x.astype(bf)
    r1 = x - hi.astype(f32)
    mid = r1.astype(bf)
    r2 = r1 - mid.astype(f32)
    return hi, mid, r2.astype(bf)


def kernel(pred, target):
    bf = jnp.bfloat16
    pred_p = jnp.pad(pred, ((0, NPAD - N), (0, 0)), constant_values=PADVAL)
    targ_p = jnp.pad(target, ((0, NPAD - N), (0, 0)), constant_values=PADVAL)

    a = pred_p[:, 0:3]
    b = targ_p[:, 0:3]
    a_sq = (a[:, 0:1] * a[:, 0:1] + a[:, 1:2] * a[:, 1:2]) + a[:, 2:3] * a[:, 2:3]
    b_sq = (b[:, 0:1] * b[:, 0:1] + b[:, 1:2] * b[:, 1:2]) + b[:, 2:3] * b[:, 2:3]
    ah, am, al = _split3(a_sq)
    bh, bm, bl = _split3(b_sq)
    one = jnp.ones((NPAD, 1), bf)
    zero7 = jnp.zeros((NPAD, 7), bf)
    amat = jnp.concatenate(
        [(a * -2.0).astype(bf), ah, am, al, one, one, one, zero7],
        axis=1)
    bmat = jnp.concatenate(
        [b.astype(bf), one, one, one, bh, bm, bl, zero7], axis=1).T

    rowidx, colidx = _knn_call(amat, bmat)

    tab_t = jnp.pad(targ_p, ((0, 0), (0, DROW - 4)))
    tab_p = jnp.pad(pred_p, ((0, 0), (0, DROW - 4)))
    g_t, g_p = _sc_gather_call(tab_t, rowidx.reshape(NPAD),
                               tab_p, colidx.reshape(NPAD))

    def pl2(x):
        return x.reshape(_R, _C)

    stk = jnp.stack([
        pl2(pred_p[:, 0]), pl2(pred_p[:, 1]), pl2(pred_p[:, 2]),
        pl2(pred_p[:, 3]),
        pl2(g_t[:, 0]), pl2(g_t[:, 1]), pl2(g_t[:, 2]), pl2(g_t[:, 3]),
        pl2(targ_p[:, 0]), pl2(targ_p[:, 1]), pl2(targ_p[:, 2]),
        pl2(g_p[:, 0]), pl2(g_p[:, 1]), pl2(g_p[:, 2]),
    ])
    total = _combine_call(stk)
    return total[0, 0]


# speedup vs baseline: 1.7527x; 1.1657x over previous
import functools

import jax
import jax.numpy as jnp
from jax import lax
from jax.experimental import pallas as pl
from jax.experimental.pallas import tpu as pltpu
from jax.experimental.pallas import tpu_sc as plsc

N = 20000
NPAD = 20480
PADVAL = 1.0e6
BI = 256
BJ = 4096
NI = NPAD // BI
NJ = NPAD // BJ

NWORKERS = 32
BPW = NPAD // NWORKERS


def _knn_body(pred_ref, ttar_ref, rowidx_ref, colidx_ref,
              rmin_scr, cmin_scr, cidx_scr):
    i = pl.program_id(0)
    j = pl.program_id(1)
    a = pred_ref[...]
    t = ttar_ref[...]
    f32 = jnp.float32
    i32 = jnp.int32
    s = lax.dot_general(a, t, (((1,), (0,)), ((), ())),
                        preferred_element_type=f32)

    v = s[:, 0:128]
    kk = jnp.zeros((BI, 128), i32)
    for k in range(1, BJ // 128):
        sk = s[:, k * 128:(k + 1) * 128]
        m = sk < v
        v = jnp.where(m, sk, v)
        kk = jnp.where(m, k, kk)
    tmin = jnp.min(v, axis=1, keepdims=True)
    lane = lax.broadcasted_iota(i32, (BI, 128), 1)
    cand = kk * 128 + lane
    jfull = jnp.where(v == tmin, cand, NPAD)
    targ = jnp.min(jfull, axis=1, keepdims=True) + j * BJ

    w = s[0:8, :]
    rr = jnp.zeros((8, BJ), i32)
    for r in range(1, BI // 8):
        sr_ = s[r * 8:(r + 1) * 8, :]
        m = sr_ < w
        w = jnp.where(m, sr_, w)
        rr = jnp.where(m, r, rr)
    cmin = jnp.min(w, axis=0, keepdims=True)
    sub = lax.broadcasted_iota(i32, (8, BJ), 0)
    cand2 = rr * 8 + sub
    ifull = jnp.where(w == cmin, cand2, NPAD)
    carg = jnp.min(ifull, axis=0, keepdims=True) + i * BI

    @pl.when(j == 0)
    def _():
        rmin_scr[...] = tmin
        rowidx_ref[...] = targ

    @pl.when(j > 0)
    def _():
        prev = rmin_scr[...]
        imp = tmin < prev
        rmin_scr[...] = jnp.where(imp, tmin, prev)
        rowidx_ref[...] = jnp.where(imp, targ, rowidx_ref[...])

    @pl.when(i == 0)
    def _():
        cmin_scr[0:1, pl.ds(j * BJ, BJ)] = cmin
        cidx_scr[0:1, pl.ds(j * BJ, BJ)] = carg

    @pl.when(i > 0)
    def _():
        pc = cmin_scr[0:1, pl.ds(j * BJ, BJ)]
        imp = cmin < pc
        cmin_scr[0:1, pl.ds(j * BJ, BJ)] = jnp.where(imp, cmin, pc)
        cidx_scr[0:1, pl.ds(j * BJ, BJ)] = jnp.where(
            imp, carg, cidx_scr[0:1, pl.ds(j * BJ, BJ)])

    @pl.when(i == NI - 1)
    def _():
        colidx_ref[...] = cidx_scr[0:1, pl.ds(j * BJ, BJ)]


_knn_call = pl.pallas_call(
    _knn_body,
    grid=(NI, NJ),
    in_specs=[
        pl.BlockSpec((BI, 16), lambda i, j: (i, 0)),
        pl.BlockSpec((16, BJ), lambda i, j: (0, j)),
    ],
    out_specs=[
        pl.BlockSpec((BI, 1), lambda i, j: (i, 0)),
        pl.BlockSpec((1, BJ), lambda i, j: (0, j)),
    ],
    out_shape=[
        jax.ShapeDtypeStruct((NPAD, 1), jnp.int32),
        jax.ShapeDtypeStruct((1, NPAD), jnp.int32),
    ],
    scratch_shapes=[
        pltpu.VMEM((BI, 1), jnp.float32),
        pltpu.VMEM((1, NPAD), jnp.float32),
        pltpu.VMEM((1, NPAD), jnp.int32),
    ],
    compiler_params=pltpu.CompilerParams(
        dimension_semantics=("arbitrary", "arbitrary")),
)


CHUNK_SC = 128
DROW = 16


def _sc_gather_body(tab_t, idx_t, tab_p, idx_p, out_t, out_p,
                    idx_v, rows_v, sem):
    wid = lax.axis_index("s") * 2 + lax.axis_index("c")
    base = wid * BPW
    for c in range(BPW // CHUNK_SC):
        off = base + c * CHUNK_SC
        pltpu.sync_copy(idx_t.at[pl.ds(off, CHUNK_SC)], idx_v)
        pltpu.async_copy(tab_t.at[idx_v], rows_v, sem).wait()
        pltpu.sync_copy(rows_v, out_t.at[pl.ds(off, CHUNK_SC)])
    for c in range(BPW // CHUNK_SC):
        off = base + c * CHUNK_SC
        pltpu.sync_copy(idx_p.at[pl.ds(off, CHUNK_SC)], idx_v)
        pltpu.async_copy(tab_p.at[idx_v], rows_v, sem).wait()
        pltpu.sync_copy(rows_v, out_p.at[pl.ds(off, CHUNK_SC)])


def _sc_gather_call(tab_t, idx_t, tab_p, idx_p):
    run = functools.partial(
        pl.kernel,
        mesh=plsc.VectorSubcoreMesh(core_axis_name="c", subcore_axis_name="s"),
        out_type=[
            jax.ShapeDtypeStruct((NPAD, DROW), jnp.float32),
            jax.ShapeDtypeStruct((NPAD, DROW), jnp.float32),
        ],
        scratch_types=[
            pltpu.VMEM((CHUNK_SC,), jnp.int32),
            pltpu.VMEM((CHUNK_SC, DROW), jnp.float32),
            pltpu.SemaphoreType.DMA,
        ],
        compiler_params=pltpu.CompilerParams(use_tc_tiling_on_sc=False),
    )(_sc_gather_body)
    return run(tab_t, idx_t, tab_p, idx_p)


_R, _C = NPAD // 128, 128
_NPLANES = 14


def _combine_body(stk_ref, out_ref):
    def plane(k):
        return stk_ref[k]

    p0, p1, p2, pint = plane(0), plane(1), plane(2), plane(3)
    g0, g1, g2, gint = plane(4), plane(5), plane(6), plane(7)
    t0, t1, t2 = plane(8), plane(9), plane(10)
    h0, h1, h2 = plane(11), plane(12), plane(13)
    r = lax.broadcasted_iota(jnp.int32, (_R, _C), 0)
    c = lax.broadcasted_iota(jnp.int32, (_R, _C), 1)
    valid = (r * _C + c) < N
    d1 = jnp.sqrt(((p0 - g0) * (p0 - g0) + (p1 - g1) * (p1 - g1))
                  + (p2 - g2) * (p2 - g2))
    d2 = jnp.sqrt(((t0 - h0) * (t0 - h0) + (t1 - h1) * (t1 - h1))
                  + (t2 - h2) * (t2 - h2))
    s1 = jnp.sum(jnp.where(valid, d1, 0.0))
    s2 = jnp.sum(jnp.where(valid, d2, 0.0))
    si = jnp.sum(jnp.where(valid, (pint - gint) * (pint - gint), 0.0))
    total = (s1 + s2) / N + 0.5 * (si / N)
    out_ref[...] = jnp.reshape(total, (1, 1))


_combine_call = pl.pallas_call(
    _combine_body,
    out_shape=jax.ShapeDtypeStruct((1, 1), jnp.float32),
)


def _split3(x):
    bf = jnp.bfloat16
    f32 = jnp.float32
    hi = x.astype(bf)
    r1 = x - hi.astype(f32)
    mid = r1.astype(bf)
    r2 = r1 - mid.astype(f32)
    return hi, mid, r2.astype(bf)


def kernel(pred, target):
    bf = jnp.bfloat16
    pred_p = jnp.pad(pred, ((0, NPAD - N), (0, 0)), constant_values=PADVAL)
    targ_p = jnp.pad(target, ((0, NPAD - N), (0, 0)), constant_values=PADVAL)

    a = pred_p[:, 0:3]
    b = targ_p[:, 0:3]
    a_sq = (a[:, 0:1] * a[:, 0:1] + a[:, 1:2] * a[:, 1:2]) + a[:, 2:3] * a[:, 2:3]
    b_sq = (b[:, 0:1] * b[:, 0:1] + b[:, 1:2] * b[:, 1:2]) + b[:, 2:3] * b[:, 2:3]
    ah, am, al = _split3(a_sq)
    bh, bm, bl = _split3(b_sq)
    one = jnp.ones((NPAD, 1), bf)
    zero7 = jnp.zeros((NPAD, 7), bf)
    amat = jnp.concatenate(
        [(a * -2.0).astype(bf), ah, am, al, one, one, one, zero7],
        axis=1)
    bmat = jnp.concatenate(
        [b.astype(bf), one, one, one, bh, bm, bl, zero7], axis=1).T

    rowidx, colidx = _knn_call(amat, bmat)

    tab_t = jnp.pad(targ_p, ((0, 0), (0, DROW - 4)))
    tab_p = jnp.pad(pred_p, ((0, 0), (0, DROW - 4)))
    g_t, g_p = _sc_gather_call(tab_t, rowidx.reshape(NPAD),
                               tab_p, colidx.reshape(NPAD))

    def pl2(x):
        return x.reshape(_R, _C)

    stk = jnp.stack([
        pl2(pred_p[:, 0]), pl2(pred_p[:, 1]), pl2(pred_p[:, 2]),
        pl2(pred_p[:, 3]),
        pl2(g_t[:, 0]), pl2(g_t[:, 1]), pl2(g_t[:, 2]), pl2(g_t[:, 3]),
        pl2(targ_p[:, 0]), pl2(targ_p[:, 1]), pl2(targ_p[:, 2]),
        pl2(g_p[:, 0]), pl2(g_p[:, 1]), pl2(g_p[:, 2]),
    ])
    total = _combine_call(stk)
    return total[0, 0]


# speedup vs baseline: 1.9116x; 1.0907x over previous
import functools

import jax
import jax.numpy as jnp
from jax import lax
from jax.experimental import pallas as pl
from jax.experimental.pallas import tpu as pltpu
from jax.experimental.pallas import tpu_sc as plsc

N = 20000
NPAD = 20480
PADVAL = 1.0e6
BI = 256
BJ = 10240
NI = NPAD // BI
NJ = NPAD // BJ

NWORKERS = 32
BPW = NPAD // NWORKERS


def _knn_body(pred_ref, ttar_ref, rowidx_ref, colidx_ref,
              rmin_scr, cmin_scr, cidx_scr):
    i = pl.program_id(0)
    j = pl.program_id(1)
    a = pred_ref[...]
    t = ttar_ref[...]
    f32 = jnp.float32
    i32 = jnp.int32
    s = lax.dot_general(a, t, (((1,), (0,)), ((), ())),
                        preferred_element_type=f32)

    v = s[:, 0:128]
    kk = jnp.zeros((BI, 128), i32)
    for k in range(1, BJ // 128):
        sk = s[:, k * 128:(k + 1) * 128]
        m = sk < v
        v = jnp.where(m, sk, v)
        kk = jnp.where(m, k, kk)
    tmin = jnp.min(v, axis=1, keepdims=True)
    lane = lax.broadcasted_iota(i32, (BI, 128), 1)
    cand = kk * 128 + lane
    jfull = jnp.where(v == tmin, cand, NPAD)
    targ = jnp.min(jfull, axis=1, keepdims=True) + j * BJ

    w = s[0:8, :]
    rr = jnp.zeros((8, BJ), i32)
    for r in range(1, BI // 8):
        sr_ = s[r * 8:(r + 1) * 8, :]
        m = sr_ < w
        w = jnp.where(m, sr_, w)
        rr = jnp.where(m, r, rr)
    cmin = jnp.min(w, axis=0, keepdims=True)
    sub = lax.broadcasted_iota(i32, (8, BJ), 0)
    cand2 = rr * 8 + sub
    ifull = jnp.where(w == cmin, cand2, NPAD)
    carg = jnp.min(ifull, axis=0, keepdims=True) + i * BI

    @pl.when(j == 0)
    def _():
        rmin_scr[...] = tmin
        rowidx_ref[...] = targ

    @pl.when(j > 0)
    def _():
        prev = rmin_scr[...]
        imp = tmin < prev
        rmin_scr[...] = jnp.where(imp, tmin, prev)
        rowidx_ref[...] = jnp.where(imp, targ, rowidx_ref[...])

    @pl.when(i == 0)
    def _():
        cmin_scr[0:1, pl.ds(j * BJ, BJ)] = cmin
        cidx_scr[0:1, pl.ds(j * BJ, BJ)] = carg

    @pl.when(i > 0)
    def _():
        pc = cmin_scr[0:1, pl.ds(j * BJ, BJ)]
        imp = cmin < pc
        cmin_scr[0:1, pl.ds(j * BJ, BJ)] = jnp.where(imp, cmin, pc)
        cidx_scr[0:1, pl.ds(j * BJ, BJ)] = jnp.where(
            imp, carg, cidx_scr[0:1, pl.ds(j * BJ, BJ)])

    @pl.when(i == NI - 1)
    def _():
        colidx_ref[...] = cidx_scr[0:1, pl.ds(j * BJ, BJ)]


_knn_call = pl.pallas_call(
    _knn_body,
    grid=(NI, NJ),
    in_specs=[
        pl.BlockSpec((BI, 16), lambda i, j: (i, 0)),
        pl.BlockSpec((16, BJ), lambda i, j: (0, j)),
    ],
    out_specs=[
        pl.BlockSpec((BI, 1), lambda i, j: (i, 0)),
        pl.BlockSpec((1, BJ), lambda i, j: (0, j)),
    ],
    out_shape=[
        jax.ShapeDtypeStruct((NPAD, 1), jnp.int32),
        jax.ShapeDtypeStruct((1, NPAD), jnp.int32),
    ],
    scratch_shapes=[
        pltpu.VMEM((BI, 1), jnp.float32),
        pltpu.VMEM((1, NPAD), jnp.float32),
        pltpu.VMEM((1, NPAD), jnp.int32),
    ],
    compiler_params=pltpu.CompilerParams(
        dimension_semantics=("arbitrary", "arbitrary")),
)


CHUNK_SC = 128
DROW = 16


def _sc_gather_body(tab_t, idx_t, tab_p, idx_p, out_t, out_p,
                    idx_v, rows_v, sem):
    wid = lax.axis_index("s") * 2 + lax.axis_index("c")
    base = wid * BPW
    for c in range(BPW // CHUNK_SC):
        off = base + c * CHUNK_SC
        pltpu.sync_copy(idx_t.at[pl.ds(off, CHUNK_SC)], idx_v)
        pltpu.async_copy(tab_t.at[idx_v], rows_v, sem).wait()
        pltpu.sync_copy(rows_v, out_t.at[pl.ds(off, CHUNK_SC)])
    for c in range(BPW // CHUNK_SC):
        off = base + c * CHUNK_SC
        pltpu.sync_copy(idx_p.at[pl.ds(off, CHUNK_SC)], idx_v)
        pltpu.async_copy(tab_p.at[idx_v], rows_v, sem).wait()
        pltpu.sync_copy(rows_v, out_p.at[pl.ds(off, CHUNK_SC)])


def _sc_gather_call(tab_t, idx_t, tab_p, idx_p):
    run = functools.partial(
        pl.kernel,
        mesh=plsc.VectorSubcoreMesh(core_axis_name="c", subcore_axis_name="s"),
        out_type=[
            jax.ShapeDtypeStruct((NPAD, DROW), jnp.float32),
            jax.ShapeDtypeStruct((NPAD, DROW), jnp.float32),
        ],
        scratch_types=[
            pltpu.VMEM((CHUNK_SC,), jnp.int32),
            pltpu.VMEM((CHUNK_SC, DROW), jnp.float32),
            pltpu.SemaphoreType.DMA,
        ],
        compiler_params=pltpu.CompilerParams(use_tc_tiling_on_sc=False),
    )(_sc_gather_body)
    return run(tab_t, idx_t, tab_p, idx_p)


_R, _C = NPAD // 128, 128
_NPLANES = 14


def _combine_body(stk_ref, out_ref):
    def plane(k):
        return stk_ref[k]

    p0, p1, p2, pint = plane(0), plane(1), plane(2), plane(3)
    g0, g1, g2, gint = plane(4), plane(5), plane(6), plane(7)
    t0, t1, t2 = plane(8), plane(9), plane(10)
    h0, h1, h2 = plane(11), plane(12), plane(13)
    r = lax.broadcasted_iota(jnp.int32, (_R, _C), 0)
    c = lax.broadcasted_iota(jnp.int32, (_R, _C), 1)
    valid = (r * _C + c) < N
    d1 = jnp.sqrt(((p0 - g0) * (p0 - g0) + (p1 - g1) * (p1 - g1))
                  + (p2 - g2) * (p2 - g2))
    d2 = jnp.sqrt(((t0 - h0) * (t0 - h0) + (t1 - h1) * (t1 - h1))
                  + (t2 - h2) * (t2 - h2))
    s1 = jnp.sum(jnp.where(valid, d1, 0.0))
    s2 = jnp.sum(jnp.where(valid, d2, 0.0))
    si = jnp.sum(jnp.where(valid, (pint - gint) * (pint - gint), 0.0))
    total = (s1 + s2) / N + 0.5 * (si / N)
    out_ref[...] = jnp.reshape(total, (1, 1))


_combine_call = pl.pallas_call(
    _combine_body,
    out_shape=jax.ShapeDtypeStruct((1, 1), jnp.float32),
)


def _split3(x):
    bf = jnp.bfloat16
    f32 = jnp.float32
    hi = x.astype(bf)
    r1 = x - hi.astype(f32)
    mid = r1.astype(bf)
    r2 = r1 - mid.astype(f32)
    return hi, mid, r2.astype(bf)


def kernel(pred, target):
    bf = jnp.bfloat16
    pred_p = jnp.pad(pred, ((0, NPAD - N), (0, 0)), constant_values=PADVAL)
    targ_p = jnp.pad(target, ((0, NPAD - N), (0, 0)), constant_values=PADVAL)

    a = pred_p[:, 0:3]
    b = targ_p[:, 0:3]
    a_sq = (a[:, 0:1] * a[:, 0:1] + a[:, 1:2] * a[:, 1:2]) + a[:, 2:3] * a[:, 2:3]
    b_sq = (b[:, 0:1] * b[:, 0:1] + b[:, 1:2] * b[:, 1:2]) + b[:, 2:3] * b[:, 2:3]
    ah, am, al = _split3(a_sq)
    bh, bm, bl = _split3(b_sq)
    one = jnp.ones((NPAD, 1), bf)
    zero7 = jnp.zeros((NPAD, 7), bf)
    amat = jnp.concatenate(
        [(a * -2.0).astype(bf), ah, am, al, one, one, one, zero7],
        axis=1)
    bmat = jnp.concatenate(
        [b.astype(bf), one, one, one, bh, bm, bl, zero7], axis=1).T

    rowidx, colidx = _knn_call(amat, bmat)

    tab_t = jnp.pad(targ_p, ((0, 0), (0, DROW - 4)))
    tab_p = jnp.pad(pred_p, ((0, 0), (0, DROW - 4)))
    g_t, g_p = _sc_gather_call(tab_t, rowidx.reshape(NPAD),
                               tab_p, colidx.reshape(NPAD))

    def pl2(x):
        return x.reshape(_R, _C)

    stk = jnp.stack([
        pl2(pred_p[:, 0]), pl2(pred_p[:, 1]), pl2(pred_p[:, 2]),
        pl2(pred_p[:, 3]),
        pl2(g_t[:, 0]), pl2(g_t[:, 1]), pl2(g_t[:, 2]), pl2(g_t[:, 3]),
        pl2(targ_p[:, 0]), pl2(targ_p[:, 1]), pl2(targ_p[:, 2]),
        pl2(g_p[:, 0]), pl2(g_p[:, 1]), pl2(g_p[:, 2]),
    ])
    total = _combine_call(stk)
    return total[0, 0]


# speedup vs baseline: 1.9798x; 1.0357x over previous
import functools

import jax
import jax.numpy as jnp
from jax import lax
from jax.experimental import pallas as pl
from jax.experimental.pallas import tpu as pltpu
from jax.experimental.pallas import tpu_sc as plsc

N = 20000
NPAD = 20480
PADVAL = 1.0e6
BI = 256
BJ = 20480
NI = NPAD // BI
NJ = NPAD // BJ

NWORKERS = 32
BPW = NPAD // NWORKERS


def _knn_body(pred_ref, ttar_ref, rowidx_ref, colidx_ref,
              rmin_scr, cmin_scr, cidx_scr):
    i = pl.program_id(0)
    j = pl.program_id(1)
    a = pred_ref[...]
    t = ttar_ref[...]
    f32 = jnp.float32
    i32 = jnp.int32
    s = lax.dot_general(a, t, (((1,), (0,)), ((), ())),
                        preferred_element_type=f32)

    v = s[:, 0:128]
    kk = jnp.zeros((BI, 128), i32)
    for k in range(1, BJ // 128):
        sk = s[:, k * 128:(k + 1) * 128]
        m = sk < v
        v = jnp.where(m, sk, v)
        kk = jnp.where(m, k, kk)
    tmin = jnp.min(v, axis=1, keepdims=True)
    lane = lax.broadcasted_iota(i32, (BI, 128), 1)
    cand = kk * 128 + lane
    jfull = jnp.where(v == tmin, cand, NPAD)
    targ = jnp.min(jfull, axis=1, keepdims=True) + j * BJ

    w = s[0:8, :]
    rr = jnp.zeros((8, BJ), i32)
    for r in range(1, BI // 8):
        sr_ = s[r * 8:(r + 1) * 8, :]
        m = sr_ < w
        w = jnp.where(m, sr_, w)
        rr = jnp.where(m, r, rr)
    cmin = jnp.min(w, axis=0, keepdims=True)
    sub = lax.broadcasted_iota(i32, (8, BJ), 0)
    cand2 = rr * 8 + sub
    ifull = jnp.where(w == cmin, cand2, NPAD)
    carg = jnp.min(ifull, axis=0, keepdims=True) + i * BI

    @pl.when(j == 0)
    def _():
        rmin_scr[...] = tmin
        rowidx_ref[...] = targ

    @pl.when(j > 0)
    def _():
        prev = rmin_scr[...]
        imp = tmin < prev
        rmin_scr[...] = jnp.where(imp, tmin, prev)
        rowidx_ref[...] = jnp.where(imp, targ, rowidx_ref[...])

    @pl.when(i == 0)
    def _():
        cmin_scr[0:1, pl.ds(j * BJ, BJ)] = cmin
        cidx_scr[0:1, pl.ds(j * BJ, BJ)] = carg

    @pl.when(i > 0)
    def _():
        pc = cmin_scr[0:1, pl.ds(j * BJ, BJ)]
        imp = cmin < pc
        cmin_scr[0:1, pl.ds(j * BJ, BJ)] = jnp.where(imp, cmin, pc)
        cidx_scr[0:1, pl.ds(j * BJ, BJ)] = jnp.where(
            imp, carg, cidx_scr[0:1, pl.ds(j * BJ, BJ)])

    @pl.when(i == NI - 1)
    def _():
        colidx_ref[...] = cidx_scr[0:1, pl.ds(j * BJ, BJ)]


_knn_call = pl.pallas_call(
    _knn_body,
    grid=(NI, NJ),
    in_specs=[
        pl.BlockSpec((BI, 16), lambda i, j: (i, 0)),
        pl.BlockSpec((16, BJ), lambda i, j: (0, j)),
    ],
    out_specs=[
        pl.BlockSpec((BI, 1), lambda i, j: (i, 0)),
        pl.BlockSpec((1, BJ), lambda i, j: (0, j)),
    ],
    out_shape=[
        jax.ShapeDtypeStruct((NPAD, 1), jnp.int32),
        jax.ShapeDtypeStruct((1, NPAD), jnp.int32),
    ],
    scratch_shapes=[
        pltpu.VMEM((BI, 1), jnp.float32),
        pltpu.VMEM((1, NPAD), jnp.float32),
        pltpu.VMEM((1, NPAD), jnp.int32),
    ],
    compiler_params=pltpu.CompilerParams(
        dimension_semantics=("arbitrary", "arbitrary")),
)


CHUNK_SC = 128
DROW = 16


def _sc_gather_body(tab_t, idx_t, tab_p, idx_p, out_t, out_p,
                    idx_v, rows_v, sem):
    wid = lax.axis_index("s") * 2 + lax.axis_index("c")
    base = wid * BPW
    for c in range(BPW // CHUNK_SC):
        off = base + c * CHUNK_SC
        pltpu.sync_copy(idx_t.at[pl.ds(off, CHUNK_SC)], idx_v)
        pltpu.async_copy(tab_t.at[idx_v], rows_v, sem).wait()
        pltpu.sync_copy(rows_v, out_t.at[pl.ds(off, CHUNK_SC)])
    for c in range(BPW // CHUNK_SC):
        off = base + c * CHUNK_SC
        pltpu.sync_copy(idx_p.at[pl.ds(off, CHUNK_SC)], idx_v)
        pltpu.async_copy(tab_p.at[idx_v], rows_v, sem).wait()
        pltpu.sync_copy(rows_v, out_p.at[pl.ds(off, CHUNK_SC)])


def _sc_gather_call(tab_t, idx_t, tab_p, idx_p):
    run = functools.partial(
        pl.kernel,
        mesh=plsc.VectorSubcoreMesh(core_axis_name="c", subcore_axis_name="s"),
        out_type=[
            jax.ShapeDtypeStruct((NPAD, DROW), jnp.float32),
            jax.ShapeDtypeStruct((NPAD, DROW), jnp.float32),
        ],
        scratch_types=[
            pltpu.VMEM((CHUNK_SC,), jnp.int32),
            pltpu.VMEM((CHUNK_SC, DROW), jnp.float32),
            pltpu.SemaphoreType.DMA,
        ],
        compiler_params=pltpu.CompilerParams(use_tc_tiling_on_sc=False),
    )(_sc_gather_body)
    return run(tab_t, idx_t, tab_p, idx_p)


_R, _C = NPAD // 128, 128
_NPLANES = 14


def _combine_body(stk_ref, out_ref):
    def plane(k):
        return stk_ref[k]

    p0, p1, p2, pint = plane(0), plane(1), plane(2), plane(3)
    g0, g1, g2, gint = plane(4), plane(5), plane(6), plane(7)
    t0, t1, t2 = plane(8), plane(9), plane(10)
    h0, h1, h2 = plane(11), plane(12), plane(13)
    r = lax.broadcasted_iota(jnp.int32, (_R, _C), 0)
    c = lax.broadcasted_iota(jnp.int32, (_R, _C), 1)
    valid = (r * _C + c) < N
    d1 = jnp.sqrt(((p0 - g0) * (p0 - g0) + (p1 - g1) * (p1 - g1))
                  + (p2 - g2) * (p2 - g2))
    d2 = jnp.sqrt(((t0 - h0) * (t0 - h0) + (t1 - h1) * (t1 - h1))
                  + (t2 - h2) * (t2 - h2))
    s1 = jnp.sum(jnp.where(valid, d1, 0.0))
    s2 = jnp.sum(jnp.where(valid, d2, 0.0))
    si = jnp.sum(jnp.where(valid, (pint - gint) * (pint - gint), 0.0))
    total = (s1 + s2) / N + 0.5 * (si / N)
    out_ref[...] = jnp.reshape(total, (1, 1))


_combine_call = pl.pallas_call(
    _combine_body,
    out_shape=jax.ShapeDtypeStruct((1, 1), jnp.float32),
)


def _split3(x):
    bf = jnp.bfloat16
    f32 = jnp.float32
    hi = x.astype(bf)
    r1 = x - hi.astype(f32)
    mid = r1.astype(bf)
    r2 = r1 - mid.astype(f32)
    return hi, mid, r2.astype(bf)


def kernel(pred, target):
    bf = jnp.bfloat16
    pred_p = jnp.pad(pred, ((0, NPAD - N), (0, 0)), constant_values=PADVAL)
    targ_p = jnp.pad(target, ((0, NPAD - N), (0, 0)), constant_values=PADVAL)

    a = pred_p[:, 0:3]
    b = targ_p[:, 0:3]
    a_sq = (a[:, 0:1] * a[:, 0:1] + a[:, 1:2] * a[:, 1:2]) + a[:, 2:3] * a[:, 2:3]
    b_sq = (b[:, 0:1] * b[:, 0:1] + b[:, 1:2] * b[:, 1:2]) + b[:, 2:3] * b[:, 2:3]
    ah, am, al = _split3(a_sq)
    bh, bm, bl = _split3(b_sq)
    one = jnp.ones((NPAD, 1), bf)
    zero7 = jnp.zeros((NPAD, 7), bf)
    amat = jnp.concatenate(
        [(a * -2.0).astype(bf), ah, am, al, one, one, one, zero7],
        axis=1)
    bmat = jnp.concatenate(
        [b.astype(bf), one, one, one, bh, bm, bl, zero7], axis=1).T

    rowidx, colidx = _knn_call(amat, bmat)

    tab_t = jnp.pad(targ_p, ((0, 0), (0, DROW - 4)))
    tab_p = jnp.pad(pred_p, ((0, 0), (0, DROW - 4)))
    g_t, g_p = _sc_gather_call(tab_t, rowidx.reshape(NPAD),
                               tab_p, colidx.reshape(NPAD))

    def pl2(x):
        return x.reshape(_R, _C)

    stk = jnp.stack([
        pl2(pred_p[:, 0]), pl2(pred_p[:, 1]), pl2(pred_p[:, 2]),
        pl2(pred_p[:, 3]),
        pl2(g_t[:, 0]), pl2(g_t[:, 1]), pl2(g_t[:, 2]), pl2(g_t[:, 3]),
        pl2(targ_p[:, 0]), pl2(targ_p[:, 1]), pl2(targ_p[:, 2]),
        pl2(g_p[:, 0]), pl2(g_p[:, 1]), pl2(g_p[:, 2]),
    ])
    total = _combine_call(stk)
    return total[0, 0]


# speedup vs baseline: 2.1529x; 1.0874x over previous
import functools

import jax
import jax.numpy as jnp
from jax import lax
from jax.experimental import pallas as pl
from jax.experimental.pallas import tpu as pltpu
from jax.experimental.pallas import tpu_sc as plsc

N = 20000
NPAD = 20480
PADVAL = 1.0e6
BI = 512
BJ = 20480
NI = NPAD // BI
NJ = NPAD // BJ

NWORKERS = 32
BPW = NPAD // NWORKERS


def _knn_body(pred_ref, ttar_ref, rowidx_ref, colidx_ref,
              rmin_scr, cmin_scr, cidx_scr):
    i = pl.program_id(0)
    j = pl.program_id(1)
    a = pred_ref[...]
    t = ttar_ref[...]
    f32 = jnp.float32
    i32 = jnp.int32
    s = lax.dot_general(a, t, (((1,), (0,)), ((), ())),
                        preferred_element_type=f32)

    v = s[:, 0:128]
    kk = jnp.zeros((BI, 128), i32)
    for k in range(1, BJ // 128):
        sk = s[:, k * 128:(k + 1) * 128]
        m = sk < v
        v = jnp.where(m, sk, v)
        kk = jnp.where(m, k, kk)
    tmin = jnp.min(v, axis=1, keepdims=True)
    lane = lax.broadcasted_iota(i32, (BI, 128), 1)
    cand = kk * 128 + lane
    jfull = jnp.where(v == tmin, cand, NPAD)
    targ = jnp.min(jfull, axis=1, keepdims=True) + j * BJ

    w = s[0:8, :]
    rr = jnp.zeros((8, BJ), i32)
    for r in range(1, BI // 8):
        sr_ = s[r * 8:(r + 1) * 8, :]
        m = sr_ < w
        w = jnp.where(m, sr_, w)
        rr = jnp.where(m, r, rr)
    cmin = jnp.min(w, axis=0, keepdims=True)
    sub = lax.broadcasted_iota(i32, (8, BJ), 0)
    cand2 = rr * 8 + sub
    ifull = jnp.where(w == cmin, cand2, NPAD)
    carg = jnp.min(ifull, axis=0, keepdims=True) + i * BI

    @pl.when(j == 0)
    def _():
        rmin_scr[...] = tmin
        rowidx_ref[...] = targ

    @pl.when(j > 0)
    def _():
        prev = rmin_scr[...]
        imp = tmin < prev
        rmin_scr[...] = jnp.where(imp, tmin, prev)
        rowidx_ref[...] = jnp.where(imp, targ, rowidx_ref[...])

    @pl.when(i == 0)
    def _():
        cmin_scr[0:1, pl.ds(j * BJ, BJ)] = cmin
        cidx_scr[0:1, pl.ds(j * BJ, BJ)] = carg

    @pl.when(i > 0)
    def _():
        pc = cmin_scr[0:1, pl.ds(j * BJ, BJ)]
        imp = cmin < pc
        cmin_scr[0:1, pl.ds(j * BJ, BJ)] = jnp.where(imp, cmin, pc)
        cidx_scr[0:1, pl.ds(j * BJ, BJ)] = jnp.where(
            imp, carg, cidx_scr[0:1, pl.ds(j * BJ, BJ)])

    @pl.when(i == NI - 1)
    def _():
        colidx_ref[...] = cidx_scr[0:1, pl.ds(j * BJ, BJ)]


_knn_call = pl.pallas_call(
    _knn_body,
    grid=(NI, NJ),
    in_specs=[
        pl.BlockSpec((BI, 16), lambda i, j: (i, 0)),
        pl.BlockSpec((16, BJ), lambda i, j: (0, j)),
    ],
    out_specs=[
        pl.BlockSpec((BI, 1), lambda i, j: (i, 0)),
        pl.BlockSpec((1, BJ), lambda i, j: (0, j)),
    ],
    out_shape=[
        jax.ShapeDtypeStruct((NPAD, 1), jnp.int32),
        jax.ShapeDtypeStruct((1, NPAD), jnp.int32),
    ],
    scratch_shapes=[
        pltpu.VMEM((BI, 1), jnp.float32),
        pltpu.VMEM((1, NPAD), jnp.float32),
        pltpu.VMEM((1, NPAD), jnp.int32),
    ],
    compiler_params=pltpu.CompilerParams(
        dimension_semantics=("arbitrary", "arbitrary")),
)


CHUNK_SC = 128
DROW = 16


def _sc_gather_body(tab_t, idx_t, tab_p, idx_p, out_t, out_p,
                    idx_v, rows_v, sem):
    wid = lax.axis_index("s") * 2 + lax.axis_index("c")
    base = wid * BPW
    for c in range(BPW // CHUNK_SC):
        off = base + c * CHUNK_SC
        pltpu.sync_copy(idx_t.at[pl.ds(off, CHUNK_SC)], idx_v)
        pltpu.async_copy(tab_t.at[idx_v], rows_v, sem).wait()
        pltpu.sync_copy(rows_v, out_t.at[pl.ds(off, CHUNK_SC)])
    for c in range(BPW // CHUNK_SC):
        off = base + c * CHUNK_SC
        pltpu.sync_copy(idx_p.at[pl.ds(off, CHUNK_SC)], idx_v)
        pltpu.async_copy(tab_p.at[idx_v], rows_v, sem).wait()
        pltpu.sync_copy(rows_v, out_p.at[pl.ds(off, CHUNK_SC)])


def _sc_gather_call(tab_t, idx_t, tab_p, idx_p):
    run = functools.partial(
        pl.kernel,
        mesh=plsc.VectorSubcoreMesh(core_axis_name="c", subcore_axis_name="s"),
        out_type=[
            jax.ShapeDtypeStruct((NPAD, DROW), jnp.float32),
            jax.ShapeDtypeStruct((NPAD, DROW), jnp.float32),
        ],
        scratch_types=[
            pltpu.VMEM((CHUNK_SC,), jnp.int32),
            pltpu.VMEM((CHUNK_SC, DROW), jnp.float32),
            pltpu.SemaphoreType.DMA,
        ],
        compiler_params=pltpu.CompilerParams(use_tc_tiling_on_sc=False),
    )(_sc_gather_body)
    return run(tab_t, idx_t, tab_p, idx_p)


_R, _C = NPAD // 128, 128
_NPLANES = 14


def _combine_body(stk_ref, out_ref):
    def plane(k):
        return stk_ref[k]

    p0, p1, p2, pint = plane(0), plane(1), plane(2), plane(3)
    g0, g1, g2, gint = plane(4), plane(5), plane(6), plane(7)
    t0, t1, t2 = plane(8), plane(9), plane(10)
    h0, h1, h2 = plane(11), plane(12), plane(13)
    r = lax.broadcasted_iota(jnp.int32, (_R, _C), 0)
    c = lax.broadcasted_iota(jnp.int32, (_R, _C), 1)
    valid = (r * _C + c) < N
    d1 = jnp.sqrt(((p0 - g0) * (p0 - g0) + (p1 - g1) * (p1 - g1))
                  + (p2 - g2) * (p2 - g2))
    d2 = jnp.sqrt(((t0 - h0) * (t0 - h0) + (t1 - h1) * (t1 - h1))
                  + (t2 - h2) * (t2 - h2))
    s1 = jnp.sum(jnp.where(valid, d1, 0.0))
    s2 = jnp.sum(jnp.where(valid, d2, 0.0))
    si = jnp.sum(jnp.where(valid, (pint - gint) * (pint - gint), 0.0))
    total = (s1 + s2) / N + 0.5 * (si / N)
    out_ref[...] = jnp.reshape(total, (1, 1))


_combine_call = pl.pallas_call(
    _combine_body,
    out_shape=jax.ShapeDtypeStruct((1, 1), jnp.float32),
)


def _split3(x):
    bf = jnp.bfloat16
    f32 = jnp.float32
    hi = x.astype(bf)
    r1 = x - hi.astype(f32)
    mid = r1.astype(bf)
    r2 = r1 - mid.astype(f32)
    return hi, mid, r2.astype(bf)


def kernel(pred, target):
    bf = jnp.bfloat16
    pred_p = jnp.pad(pred, ((0, NPAD - N), (0, 0)), constant_values=PADVAL)
    targ_p = jnp.pad(target, ((0, NPAD - N), (0, 0)), constant_values=PADVAL)

    a = pred_p[:, 0:3]
    b = targ_p[:, 0:3]
    a_sq = (a[:, 0:1] * a[:, 0:1] + a[:, 1:2] * a[:, 1:2]) + a[:, 2:3] * a[:, 2:3]
    b_sq = (b[:, 0:1] * b[:, 0:1] + b[:, 1:2] * b[:, 1:2]) + b[:, 2:3] * b[:, 2:3]
    ah, am, al = _split3(a_sq)
    bh, bm, bl = _split3(b_sq)
    one = jnp.ones((NPAD, 1), bf)
    zero7 = jnp.zeros((NPAD, 7), bf)
    amat = jnp.concatenate(
        [(a * -2.0).astype(bf), ah, am, al, one, one, one, zero7],
        axis=1)
    bmat = jnp.concatenate(
        [b.astype(bf), one, one, one, bh, bm, bl, zero7], axis=1).T

    rowidx, colidx = _knn_call(amat, bmat)

    tab_t = jnp.pad(targ_p, ((0, 0), (0, DROW - 4)))
    tab_p = jnp.pad(pred_p, ((0, 0), (0, DROW - 4)))
    g_t, g_p = _sc_gather_call(tab_t, rowidx.reshape(NPAD),
                               tab_p, colidx.reshape(NPAD))

    def pl2(x):
        return x.reshape(_R, _C)

    stk = jnp.stack([
        pl2(pred_p[:, 0]), pl2(pred_p[:, 1]), pl2(pred_p[:, 2]),
        pl2(pred_p[:, 3]),
        pl2(g_t[:, 0]), pl2(g_t[:, 1]), pl2(g_t[:, 2]), pl2(g_t[:, 3]),
        pl2(targ_p[:, 0]), pl2(targ_p[:, 1]), pl2(targ_p[:, 2]),
        pl2(g_p[:, 0]), pl2(g_p[:, 1]), pl2(g_p[:, 2]),
    ])
    total = _combine_call(stk)
    return total[0, 0]
